# Initial kernel scaffold; baseline (speedup 1.0000x reference)
import jax
import jax.numpy as jnp
from jax.experimental import pallas as pl


def kernel(x, edge_index, W_gc1, b_gc1, W_res1, b_res1, gamma1, beta1, W_gc2, b_gc2, W_res2, b_res2, gamma2, beta2, W_aw, b_aw, W_p1, b_p1, gamma_p, beta_p, W_p2, b_p2):
    raise NotImplementedError("write your pallas kernel here")


# speedup vs baseline: 5.7882x; 5.7882x over previous
import functools

import jax
import jax.numpy as jnp
from jax import lax
from jax.experimental import pallas as pl
from jax.experimental.pallas import tpu as pltpu
from jax.experimental.pallas import tpu_sc as plsc

N = 10000
E = 320000
D_IN = 128
H = 64
PH = 128

NC = 2
NS = 16
NW = NC * NS
CH = 128
NCH = 79
E_PER_W = NCH * CH
E_PAD = NW * E_PER_W
N_PAD = 10240
ROWS_PER_SUB = N_PAD // NS

f32 = jnp.float32


_sc_mesh = plsc.VectorSubcoreMesh(core_axis_name="c", subcore_axis_name="s")


@functools.partial(
    pl.kernel,
    out_type=jax.ShapeDtypeStruct((NC, N_PAD, H), f32),
    mesh=_sc_mesh,
    compiler_params=pltpu.CompilerParams(use_tc_tiling_on_sc=False),
    scratch_types=[
        pltpu.VMEM((NCH, CH), jnp.int32),
        pltpu.VMEM((NCH, CH), jnp.int32),
        pltpu.VMEM((CH, H), f32),
        pltpu.VMEM((CH, H), f32),
        pltpu.VMEM_SHARED((N_PAD, H), f32),
        pltpu.SemaphoreType.DMA,
        pltpu.SemaphoreType.DMA,
    ],
)
def _sc_scatter(t_hbm, src_hbm, dst_hbm, zeros_hbm, out_hbm,
                src_v, dst_v, rows0, rows1, acc, sem0, sem1):
    cid = lax.axis_index("c")
    sid = lax.axis_index("s")
    wid = cid * NS + sid

    pltpu.sync_copy(zeros_hbm.at[pl.ds(sid * ROWS_PER_SUB, ROWS_PER_SUB)],
                    acc.at[pl.ds(sid * ROWS_PER_SUB, ROWS_PER_SUB)])
    pltpu.sync_copy(src_hbm.at[wid], src_v)
    pltpu.sync_copy(dst_hbm.at[wid], dst_v)
    plsc.subcore_barrier()

    def body1(j, _):
        pltpu.async_copy(t_hbm.at[src_v.at[j]], rows0, sem0).wait()
        pltpu.sync_copy(rows0, acc.at[dst_v.at[j]], add=True)
        return _

    lax.fori_loop(0, NCH, body1, 0, unroll=False)

    plsc.subcore_barrier()
    pltpu.sync_copy(acc.at[pl.ds(sid * ROWS_PER_SUB, ROWS_PER_SUB)],
                    out_hbm.at[cid, pl.ds(sid * ROWS_PER_SUB, ROWS_PER_SUB)])


def _dense1_body(x_ref, wg_ref, wr_ref, br_ref, t_ref, r_ref):
    xv = x_ref[...]
    t_ref[...] = jnp.dot(xv, wg_ref[...], preferred_element_type=f32)
    r_ref[...] = jnp.maximum(
        jnp.dot(xv, wr_ref[...], preferred_element_type=f32) + br_ref[...], 0.0)


_dense1 = pl.pallas_call(
    _dense1_body,
    out_shape=[jax.ShapeDtypeStruct((N_PAD, H), f32),
               jax.ShapeDtypeStruct((N_PAD, H), f32)],
)


def _dense2_body(agg_ref, r1_ref, bg_ref, g_ref, be_ref, wg2_ref, wr2_ref,
                 br2_ref, t2_ref, r2_ref):
    agg = agg_ref[0] + agg_ref[1]
    h1 = (g_ref[...] * (jnp.maximum(agg + bg_ref[...], 0.0) + r1_ref[...])
          + be_ref[...])
    t2_ref[...] = jnp.dot(h1, wg2_ref[...], preferred_element_type=f32)
    r2_ref[...] = jnp.maximum(
        jnp.dot(h1, wr2_ref[...], preferred_element_type=f32) + br2_ref[...],
        0.0)


_dense2 = pl.pallas_call(
    _dense2_body,
    out_shape=[jax.ShapeDtypeStruct((N_PAD, H), f32),
               jax.ShapeDtypeStruct((N_PAD, H), f32)],
)


def _head_body(agg_ref, r2_ref, bg_ref, g_ref, be_ref, waw_ref, baw_ref,
               wp1_ref, bp1_ref, gp_ref, bep_ref, wp2_ref, bp2_ref,
               pred_ref, gf_ref):
    agg = agg_ref[0, :N] + agg_ref[1, :N]
    h2 = (g_ref[...] * (jnp.maximum(agg + bg_ref[...], 0.0) + r2_ref[:N])
          + be_ref[...])
    logit = jnp.sum(h2 * waw_ref[...], axis=1, keepdims=True) + baw_ref[...]
    w = jax.nn.sigmoid(logit)
    h_sum = jnp.sum(h2 * w, axis=0, keepdims=True)
    h_max = jnp.max(h2, axis=0, keepdims=True)
    gf = jnp.concatenate([h_sum, h_max], axis=1)
    z = jnp.maximum(jnp.dot(gf, wp1_ref[...], preferred_element_type=f32)
                    + bp1_ref[...], 0.0)
    z = gp_ref[...] * z + bep_ref[...]
    pred = jnp.sum(z * wp2_ref[...], axis=1, keepdims=True) + bp2_ref[...]
    pred_ref[...] = jnp.broadcast_to(pred, (1, PH))
    gf_ref[...] = gf


_head = pl.pallas_call(
    _head_body,
    out_shape=[jax.ShapeDtypeStruct((1, PH), f32),
               jax.ShapeDtypeStruct((1, 2 * H), f32)],
)


def kernel(x, edge_index, W_gc1, b_gc1, W_res1, b_res1, gamma1, beta1,
           W_gc2, b_gc2, W_res2, b_res2, gamma2, beta2, W_aw, b_aw,
           W_p1, b_p1, gamma_p, beta_p, W_p2, b_p2):
    src = edge_index[0]
    dst = edge_index[1]
    pad = E_PAD - E
    src_p = jnp.concatenate(
        [src, jnp.full((pad,), N, jnp.int32)]).reshape(NW, NCH, CH)
    dst_p = jnp.concatenate(
        [dst, jnp.full((pad,), N, jnp.int32)]).reshape(NW, NCH, CH)
    x_pad = jnp.pad(x, ((0, N_PAD - N), (0, 0)))
    zeros = jnp.zeros((N_PAD, H), f32)

    r = lambda v: v.reshape(1, -1)

    t1, r1 = _dense1(x_pad, W_gc1, W_res1, r(b_res1))
    agg1 = _sc_scatter(t1, src_p, dst_p, zeros)
    t2, r2 = _dense2(agg1, r1, r(b_gc1), r(gamma1), r(beta1),
                     W_gc2, W_res2, r(b_res2))
    agg2 = _sc_scatter(t2, src_p, dst_p, zeros)
    pred_buf, gf = _head(agg2, r2, r(b_gc2), r(gamma2), r(beta2),
                         r(W_aw), r(b_aw), W_p1, r(b_p1), r(gamma_p),
                         r(beta_p), r(W_p2), r(b_p2))
    return (pred_buf[:, :1], gf)
